# baseline (speedup 1.0000x reference)
import functools

import jax
import jax.numpy as jnp
from jax.experimental import pallas as pl
from jax.experimental.pallas import tpu as pltpu


def _prep_body(r_ref, raug_ref):
    r = r_ref[:, :]
    d = r.shape[1]
    raug_ref[:, :d] = r.astype(jnp.bfloat16)
    raug_ref[:, d:] = (r * r).astype(jnp.bfloat16)


def _chamfer_body(q_ref, raug_ref, out_ref):
    q = q_ref[:, :]
    q2 = jnp.sum(q * q, axis=1)
    q_aug = jnp.concatenate(
        [q * -2.0, jnp.ones_like(q)], axis=1).astype(jnp.bfloat16)

    t = jax.lax.dot_general(
        q_aug,
        raug_ref[:, :],
        dimension_numbers=(((1,), (1,)), ((), ())),
        preferred_element_type=jnp.float32,
    )

    row_min = jnp.min(t, axis=1) + q2
    out_ref[:, :, :] = jnp.sum(row_min).reshape(1, 1, 1)


def _mean_body(p_ref, out_ref, *, q_total):
    out_ref[:, :] = jnp.sum(p_ref[:, :, :]).reshape(1, 1) / q_total


def kernel(query, ref):
    q_total, d = query.shape
    r_total, _ = ref.shape

    tile_q = 256 if q_total % 256 == 0 else q_total
    n_q_tiles = q_total // tile_q

    raug = pl.pallas_call(
        _prep_body,
        out_shape=jax.ShapeDtypeStruct((r_total, 2 * d), jnp.bfloat16),
    )(ref)

    partials = pl.pallas_call(
        _chamfer_body,
        grid=(n_q_tiles,),
        in_specs=[
            pl.BlockSpec((tile_q, d), lambda i: (i, 0)),
            pl.BlockSpec((r_total, 2 * d), lambda i: (0, 0)),
        ],
        out_specs=pl.BlockSpec((1, 1, 1), lambda i: (i, 0, 0)),
        out_shape=jax.ShapeDtypeStruct((n_q_tiles, 1, 1), jnp.float32),
        compiler_params=pltpu.CompilerParams(
            dimension_semantics=("parallel",)),
    )(query, raug)

    out = pl.pallas_call(
        functools.partial(_mean_body, q_total=float(q_total)),
        out_shape=jax.ShapeDtypeStruct((1, 1), jnp.float32),
    )(partials)
    return out[0, 0]


# speedup vs baseline: 1.1621x; 1.1621x over previous
import functools

import jax
import jax.numpy as jnp
from jax.experimental import pallas as pl
from jax.experimental.pallas import tpu as pltpu

_N_CHUNKS = 4


def _chamfer_body(q_ref, r_ref, out_ref, raug_scratch, *, n_q_tiles, q_total):
    i = pl.program_id(0)

    @pl.when(i == 0)
    def _prep():
        r = r_ref[:, :]
        d = r.shape[1]
        raug_scratch[:, :d] = r.astype(jnp.bfloat16)
        raug_scratch[:, d:] = (r * r).astype(jnp.bfloat16)

    q = q_ref[:, :]
    q2 = jnp.sum(q * q, axis=1)
    q_aug = jnp.concatenate(
        [q * -2.0, jnp.ones_like(q)], axis=1).astype(jnp.bfloat16)

    r_total = raug_scratch.shape[0]
    chunk = r_total // _N_CHUNKS
    row_min = None
    for c in range(_N_CHUNKS):
        t_c = jax.lax.dot_general(
            q_aug,
            raug_scratch[c * chunk:(c + 1) * chunk, :],
            dimension_numbers=(((1,), (1,)), ((), ())),
            preferred_element_type=jnp.float32,
        )
        m_c = jnp.min(t_c, axis=1)
        row_min = m_c if row_min is None else jnp.minimum(row_min, m_c)

    tile_sum = jnp.sum(row_min + q2).reshape(1, 1)

    @pl.when(i == 0)
    def _init():
        out_ref[:, :] = tile_sum

    @pl.when(i > 0)
    def _acc():
        out_ref[:, :] = out_ref[:, :] + tile_sum

    @pl.when(i == n_q_tiles - 1)
    def _finish():
        out_ref[:, :] = out_ref[:, :] / q_total


def kernel(query, ref):
    q_total, d = query.shape
    r_total, _ = ref.shape

    tile_q = 256 if q_total % 256 == 0 else q_total
    n_q_tiles = q_total // tile_q

    body = functools.partial(_chamfer_body, n_q_tiles=n_q_tiles,
                             q_total=float(q_total))
    out = pl.pallas_call(
        body,
        grid=(n_q_tiles,),
        in_specs=[
            pl.BlockSpec((tile_q, d), lambda i: (i, 0)),
            pl.BlockSpec((r_total, d), lambda i: (0, 0)),
        ],
        out_specs=pl.BlockSpec((1, 1), lambda i: (0, 0)),
        out_shape=jax.ShapeDtypeStruct((1, 1), jnp.float32),
        scratch_shapes=[
            pltpu.VMEM((r_total, 2 * d), jnp.bfloat16),
        ],
    )(query, ref)
    return out[0, 0]


# speedup vs baseline: 1.1631x; 1.0009x over previous
import functools

import jax
import jax.numpy as jnp
from jax.experimental import pallas as pl
from jax.experimental.pallas import tpu as pltpu


def _chamfer_body(q_ref, r_ref, out_ref, rb_scratch, r2_scratch, *,
                  n_q_tiles, q_total):
    i = pl.program_id(0)

    @pl.when(i == 0)
    def _prep():
        r = r_ref[:, :]
        d = r.shape[1]
        rb_scratch[:, :] = r.astype(jnp.bfloat16)
        rsq = (r * r).astype(jnp.bfloat16)
        r2_scratch[:, :] = jax.lax.dot_general(
            jnp.ones((8, d), jnp.bfloat16),
            rsq,
            dimension_numbers=(((1,), (1,)), ((), ())),
            preferred_element_type=jnp.float32,
        )

    q = q_ref[:, :]
    tq = q.shape[0]
    q2 = jnp.sum(q * q, axis=1)
    qm2 = (q * -2.0).astype(jnp.bfloat16)

    t = jax.lax.dot_general(
        qm2,
        rb_scratch[:, :],
        dimension_numbers=(((1,), (1,)), ((), ())),
        preferred_element_type=jnp.float32,
    )

    t3 = t.reshape(tq // 8, 8, t.shape[1])
    m = jnp.min(t3 + r2_scratch[:, :][None, :, :], axis=2)
    tile_sum = (jnp.sum(m) + jnp.sum(q2)).reshape(1, 1)

    @pl.when(i == 0)
    def _init():
        out_ref[:, :] = tile_sum

    @pl.when(i > 0)
    def _acc():
        out_ref[:, :] = out_ref[:, :] + tile_sum

    @pl.when(i == n_q_tiles - 1)
    def _finish():
        out_ref[:, :] = out_ref[:, :] / q_total


def kernel(query, ref):
    q_total, d = query.shape
    r_total, _ = ref.shape

    tile_q = 256 if q_total % 256 == 0 else q_total
    n_q_tiles = q_total // tile_q

    body = functools.partial(_chamfer_body, n_q_tiles=n_q_tiles,
                             q_total=float(q_total))
    out = pl.pallas_call(
        body,
        grid=(n_q_tiles,),
        in_specs=[
            pl.BlockSpec((tile_q, d), lambda i: (i, 0)),
            pl.BlockSpec((r_total, d), lambda i: (0, 0)),
        ],
        out_specs=pl.BlockSpec((1, 1), lambda i: (0, 0)),
        out_shape=jax.ShapeDtypeStruct((1, 1), jnp.float32),
        scratch_shapes=[
            pltpu.VMEM((r_total, d), jnp.bfloat16),
            pltpu.VMEM((8, r_total), jnp.float32),
        ],
    )(query, ref)
    return out[0, 0]


# speedup vs baseline: 1.2606x; 1.0838x over previous
import functools

import jax
import jax.numpy as jnp
from jax.experimental import pallas as pl
from jax.experimental.pallas import tpu as pltpu


def _chamfer_body(q_ref, r_ref, out_ref, rb_scratch, r2_scratch, *,
                  n_q_tiles, q_total):
    i = pl.program_id(0)

    @pl.when(i == 0)
    def _prep():
        r = r_ref[:, :]
        d = r.shape[1]
        rb_scratch[:, :] = r.astype(jnp.bfloat16)
        rsq = (r * r).astype(jnp.bfloat16)
        r2_scratch[:, :] = jax.lax.dot_general(
            jnp.ones((8, d), jnp.bfloat16),
            rsq,
            dimension_numbers=(((1,), (1,)), ((), ())),
            preferred_element_type=jnp.float32,
        )

    q = q_ref[:, :]
    tq = q.shape[0]
    q2 = jnp.sum(q * q, axis=1)
    qm2 = (q * -2.0).astype(jnp.bfloat16)

    t = jax.lax.dot_general(
        qm2,
        rb_scratch[:, :],
        dimension_numbers=(((1,), (1,)), ((), ())),
        preferred_element_type=jnp.float32,
    )

    t3 = t.reshape(tq // 8, 8, t.shape[1])
    m = jnp.min(t3 + r2_scratch[:, :][None, :, :], axis=2)
    tile_sum = (jnp.sum(m) + jnp.sum(q2)).reshape(1, 1)

    @pl.when(i == 0)
    def _init():
        out_ref[:, :] = tile_sum

    @pl.when(i > 0)
    def _acc():
        out_ref[:, :] = out_ref[:, :] + tile_sum

    @pl.when(i == n_q_tiles - 1)
    def _finish():
        out_ref[:, :] = out_ref[:, :] / q_total


def kernel(query, ref):
    q_total, d = query.shape
    r_total, _ = ref.shape

    tile_q = 512 if q_total % 512 == 0 else q_total
    n_q_tiles = q_total // tile_q

    body = functools.partial(_chamfer_body, n_q_tiles=n_q_tiles,
                             q_total=float(q_total))
    out = pl.pallas_call(
        body,
        grid=(n_q_tiles,),
        in_specs=[
            pl.BlockSpec((tile_q, d), lambda i: (i, 0)),
            pl.BlockSpec((r_total, d), lambda i: (0, 0)),
        ],
        out_specs=pl.BlockSpec((1, 1), lambda i: (0, 0)),
        out_shape=jax.ShapeDtypeStruct((1, 1), jnp.float32),
        scratch_shapes=[
            pltpu.VMEM((r_total, d), jnp.bfloat16),
            pltpu.VMEM((8, r_total), jnp.float32),
        ],
    )(query, ref)
    return out[0, 0]


# speedup vs baseline: 1.3047x; 1.0350x over previous
import functools

import jax
import jax.numpy as jnp
from jax.experimental import pallas as pl
from jax.experimental.pallas import tpu as pltpu


def _chamfer_body(q_ref, r_ref, out_ref, rb_scratch, r2_scratch, *,
                  n_q_tiles, q_total):
    i = pl.program_id(0)

    @pl.when(i == 0)
    def _prep():
        r = r_ref[:, :]
        d = r.shape[1]
        rb_scratch[:, :] = r.astype(jnp.bfloat16)
        rsq = (r * r).astype(jnp.bfloat16)
        r2_scratch[:, :] = jax.lax.dot_general(
            jnp.ones((8, d), jnp.bfloat16),
            rsq,
            dimension_numbers=(((1,), (1,)), ((), ())),
            preferred_element_type=jnp.float32,
        )

    q = q_ref[:, :]
    tq = q.shape[0]
    q2 = jnp.sum(q * q, axis=1)
    qm2 = (q * -2.0).astype(jnp.bfloat16)

    t = jax.lax.dot_general(
        qm2,
        rb_scratch[:, :],
        dimension_numbers=(((1,), (1,)), ((), ())),
        preferred_element_type=jnp.float32,
    )

    t3 = t.reshape(tq // 8, 8, t.shape[1])
    m = jnp.min(t3 + r2_scratch[:, :][None, :, :], axis=2)
    tile_sum = (jnp.sum(m) + jnp.sum(q2)).reshape(1, 1)

    @pl.when(i == 0)
    def _init():
        out_ref[:, :] = tile_sum

    @pl.when(i > 0)
    def _acc():
        out_ref[:, :] = out_ref[:, :] + tile_sum

    @pl.when(i == n_q_tiles - 1)
    def _finish():
        out_ref[:, :] = out_ref[:, :] / q_total


def kernel(query, ref):
    q_total, d = query.shape
    r_total, _ = ref.shape

    tile_q = 1024 if q_total % 1024 == 0 else q_total
    n_q_tiles = q_total // tile_q

    body = functools.partial(_chamfer_body, n_q_tiles=n_q_tiles,
                             q_total=float(q_total))
    out = pl.pallas_call(
        body,
        grid=(n_q_tiles,),
        in_specs=[
            pl.BlockSpec((tile_q, d), lambda i: (i, 0)),
            pl.BlockSpec((r_total, d), lambda i: (0, 0)),
        ],
        out_specs=pl.BlockSpec((1, 1), lambda i: (0, 0)),
        out_shape=jax.ShapeDtypeStruct((1, 1), jnp.float32),
        scratch_shapes=[
            pltpu.VMEM((r_total, d), jnp.bfloat16),
            pltpu.VMEM((8, r_total), jnp.float32),
        ],
    )(query, ref)
    return out[0, 0]
